# baseline (speedup 1.0000x reference)
import functools

import jax
import jax.numpy as jnp
from jax import lax
from jax.experimental import pallas as pl
from jax.experimental.pallas import tpu as pltpu

LANE = 128
GRP = 4


def _round_up(x, m):
    return (x + m - 1) // m * m


def _conv_kernel(x_ref, w1_ref, b1_ref, w2_ref, b2_ref, w3_ref, b3_ref,
                 w4_ref, b4_ref, o_ref, *, ws, rows):
    r1, r2, r3, r4 = rows
    cout = o_ref.shape[-1]
    shifts1 = tuple(dy * ws + dx for dy in range(2) for dx in range(2))
    acc = jnp.zeros((r1, cout), jnp.float32)
    for t, d in enumerate(shifts1):
        acc = acc + jnp.dot(x_ref[pl.ds(d, r1), :], w1_ref[t],
                            preferred_element_type=jnp.float32)
    h = jnp.maximum(acc + b1_ref[...], 0.0).astype(jnp.bfloat16)

    shifts = tuple(kh * ws + kw for kh in range(3) for kw in range(3))
    for w_ref, b_ref, r in ((w2_ref, b2_ref, r2), (w3_ref, b3_ref, r3),
                            (w4_ref, b4_ref, r4)):
        acc = jnp.zeros((r, cout), jnp.float32)
        for t, d in enumerate(shifts):
            acc = acc + jnp.dot(h[d:d + r, :], w_ref[t],
                                preferred_element_type=jnp.float32)
        h = jnp.maximum(acc + b_ref[...], 0.0).astype(jnp.bfloat16)
    o_ref[...] = h


def _head_kernel(h_ref, a_ref, fcw_ref, fcb_ref, g_ref, be_ref,
                 w1h_ref, w1a_ref, b1_ref, w2a_ref, w2b_ref, b2_ref,
                 w3_ref, b3_ref, o_ref, *, hid):
    h = jnp.dot(h_ref[...], fcw_ref[...],
                preferred_element_type=jnp.float32) + fcb_ref[...]
    mean = jnp.mean(h, axis=-1, keepdims=True)
    var = jnp.mean(jnp.square(h - mean), axis=-1, keepdims=True)
    h = ((h - mean) * lax.rsqrt(var + 1e-5) * g_ref[...]
         + be_ref[...]).astype(jnp.bfloat16)
    z1 = jnp.maximum(
        jnp.dot(h, w1h_ref[...], preferred_element_type=jnp.float32)
        + jnp.dot(a_ref[...], w1a_ref[...], preferred_element_type=jnp.float32)
        + b1_ref[...], 0.0).astype(jnp.bfloat16)
    z2 = jnp.concatenate(
        [jnp.dot(z1[:, :hid], w2a_ref[...], preferred_element_type=jnp.float32),
         jnp.dot(z1[:, hid:], w2b_ref[...], preferred_element_type=jnp.float32)],
        axis=1)
    z2 = jnp.maximum(z2 + b2_ref[...], 0.0).astype(jnp.bfloat16)
    o_ref[...] = jnp.dot(z2, w3_ref[...],
                         preferred_element_type=jnp.float32) + b3_ref[...]


def kernel(conv_w_0, conv_b_0, conv_w_1, conv_b_1, conv_w_2, conv_b_2,
           conv_w_3, conv_b_3, fc_w, fc_b, ln_g, ln_b,
           q1_w1, q1_b1, q1_w2, q1_b2, q1_w3, q1_b3,
           q2_w1, q2_b1, q2_w2, q2_b2, q2_w3, q2_b3,
           obs, action):
    f32, bf16 = jnp.float32, jnp.bfloat16
    B, C, H, W = obs.shape
    F = conv_w_0.shape[0]
    feat = fc_w.shape[0]
    hid = q1_w2.shape[0]
    A = action.shape[1]
    hh, ws = H // 2, W // 2
    Q = B // GRP
    eye = jnp.eye(GRP, dtype=f32)

    oh = [(H - 3) // 2 + 1]
    for _ in range(3):
        oh.append(oh[-1] - 2)
    rows = [0] * 4
    rows[3] = _round_up(oh[3] * ws, 8)
    for k in range(2, -1, -1):
        rows[k] = _round_up(max(oh[k] * ws, 2 * ws + 2 + rows[k + 1]), 8)
    in_rows = _round_up(max(hh * ws, ws + 1 + rows[0]), 8)

    mats1 = []
    for dy in range(2):
        for dx in range(2):
            m = jnp.zeros((4 * C, F), f32)
            for kh in range(3):
                for kw in range(3):
                    if kh // 2 == dy and kw // 2 == dx:
                        p = (kh % 2) * 2 + (kw % 2)
                        m = m.at[p * C:(p + 1) * C, :].set(
                            conv_w_0[:, :, kh, kw].T / 255.0)
            mats1.append(jnp.kron(eye, m))
    w1p = jnp.stack(mats1).astype(bf16)

    def pack_l(w):
        return jnp.stack([jnp.kron(eye, w[:, :, kh, kw].T)
                          for kh in range(3) for kw in range(3)]).astype(bf16)

    w2p, w3p, w4p = pack_l(conv_w_1), pack_l(conv_w_2), pack_l(conv_w_3)
    b1p = jnp.tile(conv_b_0, GRP)[None, :]
    b2p = jnp.tile(conv_b_1, GRP)[None, :]
    b3p = jnp.tile(conv_b_2, GRP)[None, :]
    b4p = jnp.tile(conv_b_3, GRP)[None, :]

    x = obs.astype(bf16).reshape(Q, GRP, C, hh, 2, ws, 2)
    x = x.transpose(0, 3, 5, 1, 4, 6, 2)
    x = x.reshape(Q, hh * ws, GRP * 4 * C)
    x = jnp.pad(x, ((0, 0), (0, in_rows - hh * ws), (0, 0)))

    CO = GRP * F
    conv_flops = 2 * Q * CO * (rows[0] * 4 * GRP * C * 4
                               + (rows[1] + rows[2] + rows[3]) * CO * 9)
    conv_bytes = 2 * Q * (in_rows * GRP * 4 * C + rows[3] * CO) + \
        2 * int(w1p.size + w2p.size + w3p.size + w4p.size)
    y = pl.pallas_call(
        functools.partial(_conv_kernel, ws=ws, rows=tuple(rows)),
        out_shape=jax.ShapeDtypeStruct((Q, rows[3], CO), bf16),
        grid=(Q,),
        in_specs=[
            pl.BlockSpec((None, in_rows, GRP * 4 * C), lambda q: (q, 0, 0)),
            pl.BlockSpec(w1p.shape, lambda q: (0, 0, 0)),
            pl.BlockSpec(b1p.shape, lambda q: (0, 0)),
            pl.BlockSpec(w2p.shape, lambda q: (0, 0, 0)),
            pl.BlockSpec(b2p.shape, lambda q: (0, 0)),
            pl.BlockSpec(w3p.shape, lambda q: (0, 0, 0)),
            pl.BlockSpec(b3p.shape, lambda q: (0, 0)),
            pl.BlockSpec(w4p.shape, lambda q: (0, 0, 0)),
            pl.BlockSpec(b4p.shape, lambda q: (0, 0)),
        ],
        out_specs=pl.BlockSpec((None, rows[3], CO), lambda q: (q, 0, 0)),
        compiler_params=pltpu.CompilerParams(dimension_semantics=("parallel",)),
        cost_estimate=pl.CostEstimate(flops=conv_flops, transcendentals=0,
                                      bytes_accessed=conv_bytes),
    )(x, w1p, b1p, w2p, b2p, w3p, b3p, w4p, b4p)

    o_l = oh[3]
    y = y[:, :o_l * ws, :].reshape(Q, o_l, ws, GRP, F)
    y = y.transpose(0, 3, 1, 2, 4)[:, :, :, :o_l, :]
    h = y.reshape(B, o_l * o_l * F)

    fcw = fc_w.reshape(feat, F, o_l, o_l).transpose(2, 3, 1, 0)
    fcw = fcw.reshape(o_l * o_l * F, feat).astype(bf16)

    w1_1, w1_2 = q1_w1.T, q2_w1.T
    w1h = jnp.concatenate([w1_1[:feat], w1_2[:feat]], axis=1).astype(bf16)
    w1a = jnp.concatenate([w1_1[feat:], w1_2[feat:]], axis=1).astype(bf16)
    b1 = jnp.concatenate([q1_b1, q2_b1])[None, :]
    w2a, w2b = q1_w2.T.astype(bf16), q2_w2.T.astype(bf16)
    b2 = jnp.concatenate([q1_b2, q2_b2])[None, :]
    w3 = jnp.zeros((2 * hid, LANE), f32)
    w3 = w3.at[:hid, 0].set(q1_w3[0]).at[hid:, 1].set(q2_w3[0]).astype(bf16)
    b3 = jnp.zeros((1, LANE), f32).at[0, 0].set(q1_b3[0]).at[0, 1].set(q2_b3[0])

    weights = (fcw, fc_b[None, :], ln_g[None, :], ln_b[None, :],
               w1h, w1a, b1, w2a, w2b, b2, w3, b3)
    bm = min(128, B)
    D = h.shape[1]
    head_flops = 2 * B * (D * feat + (feat + A) * 2 * hid
                          + hid * hid * 2 + 2 * hid * LANE)
    head_bytes = 2 * B * (D + A) + 4 * B * LANE + \
        2 * sum(int(w.size) for w in weights)
    q = pl.pallas_call(
        functools.partial(_head_kernel, hid=hid),
        out_shape=jax.ShapeDtypeStruct((B, LANE), f32),
        grid=(B // bm,),
        in_specs=[pl.BlockSpec((bm, D), lambda i: (i, 0)),
                  pl.BlockSpec((bm, A), lambda i: (i, 0))]
                 + [pl.BlockSpec(w.shape, lambda i, _nd=w.ndim: (0,) * _nd)
                    for w in weights],
        out_specs=pl.BlockSpec((bm, LANE), lambda i: (i, 0)),
        compiler_params=pltpu.CompilerParams(dimension_semantics=("parallel",)),
        cost_estimate=pl.CostEstimate(flops=head_flops, transcendentals=B,
                                      bytes_accessed=head_bytes),
    )(h, action.astype(bf16), *weights)

    return q[:, 0:1], q[:, 1:2]


# speedup vs baseline: 14.6550x; 14.6550x over previous
import functools

import jax
import jax.numpy as jnp
from jax import lax
from jax.experimental import pallas as pl
from jax.experimental.pallas import tpu as pltpu

LANE = 128
GRP = 4


def _round_up(x, m):
    return (x + m - 1) // m * m


def _conv_kernel(x_ref, w1_ref, b1_ref, w2_ref, b2_ref, w3_ref, b3_ref,
                 w4_ref, b4_ref, o_ref, *, ws, rows):
    r1, r2, r3, r4 = rows
    cout = o_ref.shape[-1]
    xp = jnp.concatenate([x_ref[g] for g in range(x_ref.shape[0])], axis=1)
    shifts1 = tuple(dy * ws + dx for dy in range(2) for dx in range(2))
    acc = jnp.zeros((r1, cout), jnp.float32)
    for t, d in enumerate(shifts1):
        acc = acc + jnp.dot(xp[d:d + r1, :], w1_ref[t],
                            preferred_element_type=jnp.float32)
    h = jnp.maximum(acc + b1_ref[...], 0.0).astype(jnp.bfloat16)

    shifts = tuple(kh * ws + kw for kh in range(3) for kw in range(3))
    for w_ref, b_ref, r in ((w2_ref, b2_ref, r2), (w3_ref, b3_ref, r3),
                            (w4_ref, b4_ref, r4)):
        acc = jnp.zeros((r, cout), jnp.float32)
        for t, d in enumerate(shifts):
            acc = acc + jnp.dot(h[d:d + r, :], w_ref[t],
                                preferred_element_type=jnp.float32)
        h = jnp.maximum(acc + b_ref[...], 0.0).astype(jnp.bfloat16)
    o_ref[...] = h


def _head_kernel(h_ref, a_ref, fcw_ref, fcb_ref, g_ref, be_ref,
                 w1h_ref, w1a_ref, b1_ref, w2a_ref, w2b_ref, b2_ref,
                 w3_ref, b3_ref, o_ref, *, hid):
    h = jnp.dot(h_ref[...], fcw_ref[...],
                preferred_element_type=jnp.float32) + fcb_ref[...]
    mean = jnp.mean(h, axis=-1, keepdims=True)
    var = jnp.mean(jnp.square(h - mean), axis=-1, keepdims=True)
    h = ((h - mean) * lax.rsqrt(var + 1e-5) * g_ref[...]
         + be_ref[...]).astype(jnp.bfloat16)
    z1 = jnp.maximum(
        jnp.dot(h, w1h_ref[...], preferred_element_type=jnp.float32)
        + jnp.dot(a_ref[...], w1a_ref[...], preferred_element_type=jnp.float32)
        + b1_ref[...], 0.0).astype(jnp.bfloat16)
    z2 = jnp.concatenate(
        [jnp.dot(z1[:, :hid], w2a_ref[...], preferred_element_type=jnp.float32),
         jnp.dot(z1[:, hid:], w2b_ref[...], preferred_element_type=jnp.float32)],
        axis=1)
    z2 = jnp.maximum(z2 + b2_ref[...], 0.0).astype(jnp.bfloat16)
    o_ref[...] = jnp.dot(z2, w3_ref[...],
                         preferred_element_type=jnp.float32) + b3_ref[...]


def kernel(conv_w_0, conv_b_0, conv_w_1, conv_b_1, conv_w_2, conv_b_2,
           conv_w_3, conv_b_3, fc_w, fc_b, ln_g, ln_b,
           q1_w1, q1_b1, q1_w2, q1_b2, q1_w3, q1_b3,
           q2_w1, q2_b1, q2_w2, q2_b2, q2_w3, q2_b3,
           obs, action):
    f32, bf16 = jnp.float32, jnp.bfloat16
    B, C, H, W = obs.shape
    F = conv_w_0.shape[0]
    feat = fc_w.shape[0]
    hid = q1_w2.shape[0]
    A = action.shape[1]
    hh, ws = H // 2, W // 2
    Q = B // GRP
    eye = jnp.eye(GRP, dtype=f32)

    oh = [(H - 3) // 2 + 1]
    for _ in range(3):
        oh.append(oh[-1] - 2)
    rows = [0] * 4
    rows[3] = _round_up(oh[3] * ws, 8)
    for k in range(2, -1, -1):
        rows[k] = _round_up(max(oh[k] * ws, 2 * ws + 2 + rows[k + 1]), 8)
    in_rows = _round_up(max(hh * ws, ws + 1 + rows[0]), 8)

    mats1 = []
    for dy in range(2):
        for dx in range(2):
            m = jnp.zeros((4 * C, F), f32)
            for kh in range(3):
                for kw in range(3):
                    if kh // 2 == dy and kw // 2 == dx:
                        p = (kh % 2) * 2 + (kw % 2)
                        m = m.at[p * C:(p + 1) * C, :].set(
                            conv_w_0[:, :, kh, kw].T / 255.0)
            mats1.append(jnp.kron(eye, m))
    w1p = jnp.stack(mats1).astype(bf16)

    def pack_l(w):
        return jnp.stack([jnp.kron(eye, w[:, :, kh, kw].T)
                          for kh in range(3) for kw in range(3)]).astype(bf16)

    w2p, w3p, w4p = pack_l(conv_w_1), pack_l(conv_w_2), pack_l(conv_w_3)
    b1p = jnp.tile(conv_b_0, GRP)[None, :]
    b2p = jnp.tile(conv_b_1, GRP)[None, :]
    b3p = jnp.tile(conv_b_2, GRP)[None, :]
    b4p = jnp.tile(conv_b_3, GRP)[None, :]

    x = obs.reshape(B, C, hh, 2, ws, 2)
    x = x.transpose(0, 2, 4, 3, 5, 1)
    x = x.reshape(B, hh * ws, 4 * C)
    x = jnp.pad(x, ((0, 0), (0, in_rows - hh * ws), (0, 0))).astype(bf16)

    CO = GRP * F
    conv_flops = 2 * Q * CO * (rows[0] * 4 * GRP * C * 4
                               + (rows[1] + rows[2] + rows[3]) * CO * 9)
    conv_bytes = 2 * Q * (in_rows * GRP * 4 * C + rows[3] * CO) + \
        2 * int(w1p.size + w2p.size + w3p.size + w4p.size)
    y = pl.pallas_call(
        functools.partial(_conv_kernel, ws=ws, rows=tuple(rows)),
        out_shape=jax.ShapeDtypeStruct((Q, rows[3], CO), bf16),
        grid=(Q,),
        in_specs=[
            pl.BlockSpec((GRP, in_rows, 4 * C), lambda q: (q, 0, 0)),
            pl.BlockSpec(w1p.shape, lambda q: (0, 0, 0)),
            pl.BlockSpec(b1p.shape, lambda q: (0, 0)),
            pl.BlockSpec(w2p.shape, lambda q: (0, 0, 0)),
            pl.BlockSpec(b2p.shape, lambda q: (0, 0)),
            pl.BlockSpec(w3p.shape, lambda q: (0, 0, 0)),
            pl.BlockSpec(b3p.shape, lambda q: (0, 0)),
            pl.BlockSpec(w4p.shape, lambda q: (0, 0, 0)),
            pl.BlockSpec(b4p.shape, lambda q: (0, 0)),
        ],
        out_specs=pl.BlockSpec((None, rows[3], CO), lambda q: (q, 0, 0)),
        compiler_params=pltpu.CompilerParams(dimension_semantics=("parallel",)),
        cost_estimate=pl.CostEstimate(flops=conv_flops, transcendentals=0,
                                      bytes_accessed=conv_bytes),
    )(x, w1p, b1p, w2p, b2p, w3p, b3p, w4p, b4p)

    o_l = oh[3]
    y = y[:, :o_l * ws, :].reshape(Q, o_l, ws, GRP, F)
    y = y.transpose(0, 3, 1, 2, 4)[:, :, :, :o_l, :]
    h = y.reshape(B, o_l * o_l * F)

    fcw = fc_w.reshape(feat, F, o_l, o_l).transpose(2, 3, 1, 0)
    fcw = fcw.reshape(o_l * o_l * F, feat).astype(bf16)

    w1_1, w1_2 = q1_w1.T, q2_w1.T
    w1h = jnp.concatenate([w1_1[:feat], w1_2[:feat]], axis=1).astype(bf16)
    w1a = jnp.concatenate([w1_1[feat:], w1_2[feat:]], axis=1).astype(bf16)
    b1 = jnp.concatenate([q1_b1, q2_b1])[None, :]
    w2a, w2b = q1_w2.T.astype(bf16), q2_w2.T.astype(bf16)
    b2 = jnp.concatenate([q1_b2, q2_b2])[None, :]
    w3 = jnp.zeros((2 * hid, LANE), f32)
    w3 = w3.at[:hid, 0].set(q1_w3[0]).at[hid:, 1].set(q2_w3[0]).astype(bf16)
    b3 = jnp.zeros((1, LANE), f32).at[0, 0].set(q1_b3[0]).at[0, 1].set(q2_b3[0])

    weights = (fcw, fc_b[None, :], ln_g[None, :], ln_b[None, :],
               w1h, w1a, b1, w2a, w2b, b2, w3, b3)
    bm = min(128, B)
    D = h.shape[1]
    head_flops = 2 * B * (D * feat + (feat + A) * 2 * hid
                          + hid * hid * 2 + 2 * hid * LANE)
    head_bytes = 2 * B * (D + A) + 4 * B * LANE + \
        2 * sum(int(w.size) for w in weights)
    q = pl.pallas_call(
        functools.partial(_head_kernel, hid=hid),
        out_shape=jax.ShapeDtypeStruct((B, LANE), f32),
        grid=(B // bm,),
        in_specs=[pl.BlockSpec((bm, D), lambda i: (i, 0)),
                  pl.BlockSpec((bm, A), lambda i: (i, 0))]
                 + [pl.BlockSpec(w.shape, lambda i, _nd=w.ndim: (0,) * _nd)
                    for w in weights],
        out_specs=pl.BlockSpec((bm, LANE), lambda i: (i, 0)),
        compiler_params=pltpu.CompilerParams(dimension_semantics=("parallel",)),
        cost_estimate=pl.CostEstimate(flops=head_flops, transcendentals=B,
                                      bytes_accessed=head_bytes),
    )(h, action.astype(bf16), *weights)

    return q[:, 0:1], q[:, 1:2]
